# baseline (speedup 1.0000x reference)
import functools

import jax
import jax.numpy as jnp
from jax import lax
from jax.experimental import pallas as pl
from jax.experimental.pallas import tpu as pltpu
from jax.experimental.pallas import tpu_sc as plsc

B = 32
ROWS, COLS = 2048, 1024
N = ROWS * COLS
TOPK = 16

SC_L = 16
SC_NC, SC_NS = 2, 16
NW = SC_NC * SC_NS
CR = 16
NBUF = 2
NCHUNK = ROWS // CR

_mesh = plsc.VectorSubcoreMesh(core_axis_name="c", subcore_axis_name="s",
                               num_cores=SC_NC, num_subcores=SC_NS)


@functools.partial(
    pl.kernel,
    out_type=jax.ShapeDtypeStruct((NW, SC_L), jnp.float32),
    mesh=_mesh,
    compiler_params=pltpu.CompilerParams(use_tc_tiling_on_sc=True),
    scratch_types=[
        pltpu.VMEM((NBUF, CR, COLS), jnp.float32),
        pltpu.VMEM((NBUF, CR, COLS), jnp.float32),
        pltpu.VMEM((SC_L,), jnp.float32),
        pltpu.SemaphoreType.DMA,
        pltpu.SemaphoreType.DMA,
    ],
)
def _sc_reduce(o_hbm, l_hbm, out_hbm, obuf, lbuf, accv, sem0, sem1):
    sems = (sem0, sem1)
    wid = lax.axis_index("s") * SC_NC + lax.axis_index("c")
    base = wid * ROWS

    def _start(i, slot):
        pltpu.async_copy(o_hbm.at[pl.ds(base + i * CR, CR)], obuf.at[slot],
                         sems[slot])
        pltpu.async_copy(l_hbm.at[pl.ds(base + i * CR, CR)], lbuf.at[slot],
                         sems[slot])

    def _wait(i, slot):
        pltpu.make_async_copy(o_hbm.at[pl.ds(base + i * CR, CR)],
                              obuf.at[slot], sems[slot]).wait()
        pltpu.make_async_copy(l_hbm.at[pl.ds(base + i * CR, CR)],
                              lbuf.at[slot], sems[slot]).wait()

    for b in range(NBUF):
        _start(b, b)
    accv[...] = jnp.zeros((SC_L,), jnp.float32)

    @pl.loop(0, NCHUNK, step=NBUF)
    def _outer(g):
        for b in range(NBUF):
            i = g + b
            _wait(i, b)

            @pl.loop(0, COLS // SC_L, init_carry=jnp.zeros((SC_L,), jnp.float32),
                     unroll=2)
            def chunk_acc(j, acc):
                sq = []
                for r in range(CR):
                    d = (obuf.at[b][r, pl.ds(j * SC_L, SC_L)]
                         - lbuf.at[b][r, pl.ds(j * SC_L, SC_L)])
                    sq.append(d * d)
                while len(sq) > 1:
                    sq = [sq[k] + sq[k + 1] for k in range(0, len(sq), 2)]
                return acc + sq[0]

            accv[...] += chunk_acc
            nxt = i + NBUF

            @pl.when(nxt < NCHUNK)
            def _():
                _start(nxt, b)

    pltpu.sync_copy(accv, out_hbm.at[wid])


def _topk_body(acc_ref, out_ref):
    vals0 = jnp.sum(acc_ref[...], axis=1, keepdims=True) * (1.0 / N)
    ii = lax.broadcasted_iota(jnp.int32, (B, 1), 0)
    jk = lax.broadcasted_iota(jnp.int32, (1, TOPK), 1)

    def _extract(k, carry):
        vals, outr = carry
        m = jnp.max(vals)
        outr = jnp.where(jk == k, m, outr)
        first = jnp.min(jnp.where(vals == m, ii, 2 * B))
        vals = jnp.where(ii == first, -jnp.inf, vals)
        return vals, outr

    _, outr = lax.fori_loop(0, TOPK, _extract,
                            (vals0, jnp.zeros((1, TOPK), jnp.float32)))
    out_ref[...] = outr


def kernel(output, label):
    o2 = output.reshape(B * ROWS, COLS)
    l2 = label.reshape(B * ROWS, COLS)
    acc = _sc_reduce(o2, l2)
    out = pl.pallas_call(
        _topk_body,
        out_shape=jax.ShapeDtypeStruct((1, TOPK), jnp.float32),
    )(acc)
    return out[0]


# speedup vs baseline: 1.0265x; 1.0265x over previous
import functools

import jax
import jax.numpy as jnp
from jax import lax
from jax.experimental import pallas as pl
from jax.experimental.pallas import tpu as pltpu
from jax.experimental.pallas import tpu_sc as plsc

B = 32
ROWS, COLS = 2048, 1024
N = ROWS * COLS
TOPK = 16

SC_L = 16
SC_NC, SC_NS = 2, 16
NW = SC_NC * SC_NS
CR = 16
NBUF = 2
NCHUNK = ROWS // CR

_mesh = plsc.VectorSubcoreMesh(core_axis_name="c", subcore_axis_name="s",
                               num_cores=SC_NC, num_subcores=SC_NS)


@functools.partial(
    pl.kernel,
    out_type=jax.ShapeDtypeStruct((NW, SC_L), jnp.float32),
    mesh=_mesh,
    compiler_params=pltpu.CompilerParams(use_tc_tiling_on_sc=True),
    scratch_types=[
        pltpu.VMEM((NBUF, CR, COLS), jnp.float32),
        pltpu.VMEM((NBUF, CR, COLS), jnp.float32),
        pltpu.VMEM((SC_L,), jnp.float32),
        pltpu.SemaphoreType.DMA,
        pltpu.SemaphoreType.DMA,
    ],
)
def _sc_reduce(o_hbm, l_hbm, out_hbm, obuf, lbuf, accv, sem0, sem1):
    sems = (sem0, sem1)
    wid = lax.axis_index("s") * SC_NC + lax.axis_index("c")
    base = wid * ROWS

    def _start(i, slot):
        pltpu.async_copy(o_hbm.at[pl.ds(base + i * CR, CR)], obuf.at[slot],
                         sems[slot])
        pltpu.async_copy(l_hbm.at[pl.ds(base + i * CR, CR)], lbuf.at[slot],
                         sems[slot])

    def _wait(i, slot):
        pltpu.make_async_copy(o_hbm.at[pl.ds(base + i * CR, CR)],
                              obuf.at[slot], sems[slot]).wait()
        pltpu.make_async_copy(l_hbm.at[pl.ds(base + i * CR, CR)],
                              lbuf.at[slot], sems[slot]).wait()

    for b in range(NBUF):
        _start(b, b)
    accv[...] = jnp.zeros((SC_L,), jnp.float32)

    @pl.loop(0, NCHUNK, step=NBUF)
    def _outer(g):
        for b in range(NBUF):
            i = g + b
            _wait(i, b)

            @plsc.parallel_loop(0, COLS // SC_L, unroll=2,
                                carry=jnp.zeros((SC_L,), jnp.float32))
            def chunk_acc(j, acc):
                sq = []
                for r in range(CR):
                    d = (obuf.at[b][r, pl.ds(j * SC_L, SC_L)]
                         - lbuf.at[b][r, pl.ds(j * SC_L, SC_L)])
                    sq.append(d * d)
                while len(sq) > 1:
                    sq = [sq[k] + sq[k + 1] for k in range(0, len(sq), 2)]
                return acc + sq[0]

            accv[...] += chunk_acc
            nxt = i + NBUF

            @pl.when(nxt < NCHUNK)
            def _():
                _start(nxt, b)

    pltpu.sync_copy(accv, out_hbm.at[wid])


def _topk_body(acc_ref, out_ref):
    vals0 = jnp.sum(acc_ref[...], axis=1, keepdims=True) * (1.0 / N)
    ii = lax.broadcasted_iota(jnp.int32, (B, 1), 0)
    jk = lax.broadcasted_iota(jnp.int32, (1, TOPK), 1)

    def _extract(k, carry):
        vals, outr = carry
        m = jnp.max(vals)
        outr = jnp.where(jk == k, m, outr)
        first = jnp.min(jnp.where(vals == m, ii, 2 * B))
        vals = jnp.where(ii == first, -jnp.inf, vals)
        return vals, outr

    _, outr = lax.fori_loop(0, TOPK, _extract,
                            (vals0, jnp.zeros((1, TOPK), jnp.float32)))
    out_ref[...] = outr


def kernel(output, label):
    o2 = output.reshape(B * ROWS, COLS)
    l2 = label.reshape(B * ROWS, COLS)
    acc = _sc_reduce(o2, l2)
    out = pl.pallas_call(
        _topk_body,
        out_shape=jax.ShapeDtypeStruct((1, TOPK), jnp.float32),
    )(acc)
    return out[0]


# speedup vs baseline: 1.2406x; 1.2085x over previous
import functools

import jax
import jax.numpy as jnp
from jax import lax
from jax.experimental import pallas as pl
from jax.experimental.pallas import tpu as pltpu
from jax.experimental.pallas import tpu_sc as plsc

B = 32
ROWS, COLS = 2048, 1024
N = ROWS * COLS
TOPK = 16

SC_SAMPLES = 16
TC_SAMPLES = B - SC_SAMPLES

SC_L = 16
SC_NC, SC_NS = 2, 16
NW = SC_NC * SC_NS
WPS = NW // SC_SAMPLES
WROWS = ROWS // WPS
CR = 16
NBUF = 2
NCHUNK = WROWS // CR

_mesh = plsc.VectorSubcoreMesh(core_axis_name="c", subcore_axis_name="s",
                               num_cores=SC_NC, num_subcores=SC_NS)


@functools.partial(
    pl.kernel,
    out_type=jax.ShapeDtypeStruct((NW, SC_L), jnp.float32),
    mesh=_mesh,
    compiler_params=pltpu.CompilerParams(use_tc_tiling_on_sc=True),
    scratch_types=[
        pltpu.VMEM((NBUF, CR, COLS), jnp.float32),
        pltpu.VMEM((NBUF, CR, COLS), jnp.float32),
        pltpu.VMEM((SC_L,), jnp.float32),
        pltpu.SemaphoreType.DMA,
        pltpu.SemaphoreType.DMA,
    ],
)
def _sc_reduce(o_hbm, l_hbm, out_hbm, obuf, lbuf, accv, sem0, sem1):
    sems = (sem0, sem1)
    wid = lax.axis_index("s") * SC_NC + lax.axis_index("c")
    base = TC_SAMPLES * ROWS + wid * WROWS

    def _start(i, slot):
        pltpu.async_copy(o_hbm.at[pl.ds(base + i * CR, CR)], obuf.at[slot],
                         sems[slot])
        pltpu.async_copy(l_hbm.at[pl.ds(base + i * CR, CR)], lbuf.at[slot],
                         sems[slot])

    def _wait(i, slot):
        pltpu.make_async_copy(o_hbm.at[pl.ds(base + i * CR, CR)],
                              obuf.at[slot], sems[slot]).wait()
        pltpu.make_async_copy(l_hbm.at[pl.ds(base + i * CR, CR)],
                              lbuf.at[slot], sems[slot]).wait()

    for b in range(NBUF):
        _start(b, b)
    accv[...] = jnp.zeros((SC_L,), jnp.float32)

    @pl.loop(0, NCHUNK, step=NBUF)
    def _outer(g):
        for b in range(NBUF):
            i = g + b
            _wait(i, b)

            @plsc.parallel_loop(0, COLS // SC_L, unroll=2,
                                carry=jnp.zeros((SC_L,), jnp.float32))
            def chunk_acc(j, acc):
                sq = []
                for r in range(CR):
                    d = (obuf.at[b][r, pl.ds(j * SC_L, SC_L)]
                         - lbuf.at[b][r, pl.ds(j * SC_L, SC_L)])
                    sq.append(d * d)
                while len(sq) > 1:
                    sq = [sq[k] + sq[k + 1] for k in range(0, len(sq), 2)]
                return acc + sq[0]

            accv[...] += chunk_acc
            nxt = i + NBUF

            @pl.when(nxt < NCHUNK)
            def _():
                _start(nxt, b)

    pltpu.sync_copy(accv, out_hbm.at[wid])


TC_BR = 1024
TC_SPS = ROWS // TC_BR
TC_GRID = TC_SAMPLES * TC_SPS


def _tc_body(o_ref, l_ref, out_ref, acc_ref):
    step = pl.program_id(0)
    sample = step // TC_SPS

    d = o_ref[...] - l_ref[...]
    s = jnp.sum(d * d)

    @pl.when(step % TC_SPS == 0)
    def _first():
        acc_ref[sample] = s

    @pl.when(step % TC_SPS != 0)
    def _rest():
        acc_ref[sample] += s

    @pl.when(step == TC_GRID - 1)
    def _emit():
        ii = lax.broadcasted_iota(jnp.int32, (TC_SAMPLES, 1), 0)

        def _build(i, vals):
            return jnp.where(ii == i, acc_ref[i], vals)

        out_ref[...] = lax.fori_loop(0, TC_SAMPLES, _build,
                                     jnp.zeros((TC_SAMPLES, 1), jnp.float32))


def _topk_body(tc_ref, sc_ref, out_ref):
    sc_sums = jnp.sum(sc_ref[...], axis=1, keepdims=True)
    vals0 = jnp.concatenate([tc_ref[...], sc_sums], axis=0) * (1.0 / N)
    ii = lax.broadcasted_iota(jnp.int32, (B, 1), 0)
    jk = lax.broadcasted_iota(jnp.int32, (1, TOPK), 1)

    def _extract(k, carry):
        vals, outr = carry
        m = jnp.max(vals)
        outr = jnp.where(jk == k, m, outr)
        first = jnp.min(jnp.where(vals == m, ii, 2 * B))
        vals = jnp.where(ii == first, -jnp.inf, vals)
        return vals, outr

    _, outr = lax.fori_loop(0, TOPK, _extract,
                            (vals0, jnp.zeros((1, TOPK), jnp.float32)))
    out_ref[...] = outr


def kernel(output, label):
    o2 = output.reshape(B * ROWS, COLS)
    l2 = label.reshape(B * ROWS, COLS)
    sc_acc = _sc_reduce(o2, l2)
    tc_sums = pl.pallas_call(
        _tc_body,
        grid=(TC_GRID,),
        in_specs=[pl.BlockSpec((TC_BR, COLS), lambda i: (i, 0)),
                  pl.BlockSpec((TC_BR, COLS), lambda i: (i, 0))],
        out_specs=pl.BlockSpec((TC_SAMPLES, 1), lambda i: (0, 0)),
        out_shape=jax.ShapeDtypeStruct((TC_SAMPLES, 1), jnp.float32),
        scratch_shapes=[pltpu.SMEM((TC_SAMPLES,), jnp.float32)],
    )(o2, l2)
    sc_grouped = sc_acc.reshape(SC_SAMPLES, WPS * SC_L)
    out = pl.pallas_call(
        _topk_body,
        out_shape=jax.ShapeDtypeStruct((1, TOPK), jnp.float32),
    )(tc_sums, sc_grouped)
    return out[0]
